# baseline (speedup 1.0000x reference)
import functools

import jax
import jax.numpy as jnp
from jax import lax
from jax.experimental import pallas as pl
from jax.experimental.pallas import tpu as pltpu
from jax.experimental.pallas import tpu_sc as plsc

_CAPACITY = 0.8
_LANES = 16
_RADIX_BITS = 11
_RADIX = 1 << _RADIX_BITS
_DIG_MASK = _RADIX - 1
_MIN_I32 = -(2**31)


def _score_body(x_ref, w_ref, o_ref):
    s = lax.dot_general(
        w_ref[...], x_ref[...], (((1,), (1,)), ((), ())),
        preferred_element_type=jnp.float32,
    )
    o_ref[...] = 1.0 / (jnp.exp(-s) + 1.0)


def _probs_tc(x2d, W, blk):
    n, d = x2d.shape
    grid = n // blk
    out = pl.pallas_call(
        _score_body,
        grid=(grid,),
        in_specs=[
            pl.BlockSpec((blk, d), lambda i: (i, 0)),
            pl.BlockSpec((1, d), lambda i: (0, 0)),
        ],
        out_specs=pl.BlockSpec((1, blk), lambda i: (0, i)),
        out_shape=jax.ShapeDtypeStruct((1, n), jnp.float32),
    )(x2d, W)
    return out.reshape(n)


def _key_from_prob(p_chunk):
    b = lax.bitcast_convert_type(p_chunk, jnp.int32)
    u = jnp.where(b < 0, ~b, b ^ _MIN_I32)
    return ~u


def _prob_from_key(key):
    u = ~key
    b = jnp.where(u < 0, u ^ _MIN_I32, ~u)
    return lax.bitcast_convert_type(b, jnp.float32)


def _digit(key, shift):
    return lax.shift_right_logical(key, shift) & _DIG_MASK


def _make_sort_sc(batch, seq):
    info = plsc.get_sparse_core_info()
    nc = info.num_cores
    nvec = seq // _LANES
    nhist = _RADIX // _LANES
    mesh = plsc.VectorSubcoreMesh(core_axis_name="c", subcore_axis_name="s")

    def body(probs_hbm, vals_hbm, idx_hbm, pf, ka, ia, kb, ib, hist):
        wid = lax.axis_index("s") * nc + lax.axis_index("c")

        @pl.when(wid < batch)
        def _():
            base = pl.multiple_of(wid * seq, seq)
            pltpu.sync_copy(probs_hbm.at[pl.ds(base, seq)], pf)

            def run_pass(p, src_k, src_v, dst_k, dst_v):
                shift = _RADIX_BITS * p

                def zero_body(j, _):
                    hist[pl.ds(j * _LANES, _LANES)] = jnp.zeros(
                        (_LANES,), jnp.int32)
                    return 0

                lax.fori_loop(0, nhist, zero_body, 0)

                def load_key(i):
                    off = i * _LANES
                    if p == 0:
                        return _key_from_prob(pf[pl.ds(off, _LANES)])
                    return src_k[pl.ds(off, _LANES)]

                def hist_body(i, _):
                    d = _digit(load_key(i), shift)
                    c, last = plsc.scan_count(d)
                    plsc.addupdate_scatter(hist, [d], c, mask=last)
                    return 0

                lax.fori_loop(0, nvec, hist_body, 0)

                def prefix_body(j, carry):
                    h = hist[pl.ds(j * _LANES, _LANES)]
                    cs = plsc.cumsum(h)
                    hist[pl.ds(j * _LANES, _LANES)] = cs - h + carry
                    return carry + jnp.sum(h)

                lax.fori_loop(0, nhist, prefix_body, jnp.int32(0))

                def perm_body(i, _):
                    off = i * _LANES
                    key = load_key(i)
                    d = _digit(key, shift)
                    c, last = plsc.scan_count(d)
                    pos = plsc.load_gather(hist, [d]) + c - 1
                    if p == 0:
                        v = lax.iota(jnp.int32, _LANES) + off
                    else:
                        v = src_v[pl.ds(off, _LANES)]
                    if p == 2:
                        plsc.store_scatter(dst_k, [pos], _prob_from_key(key))
                    else:
                        plsc.store_scatter(dst_k, [pos], key)
                    plsc.store_scatter(dst_v, [pos], v)
                    plsc.addupdate_scatter(hist, [d], c, mask=last)
                    return 0

                lax.fori_loop(0, nvec, perm_body, 0)

            run_pass(0, pf, None, ka, ia)
            run_pass(1, ka, ia, kb, ib)
            run_pass(2, kb, ib, pf, ia)

            pltpu.sync_copy(pf, vals_hbm.at[pl.ds(base, seq)])
            pltpu.sync_copy(ia, idx_hbm.at[pl.ds(base, seq)])

    n = batch * seq
    return pl.kernel(
        body,
        out_type=(
            jax.ShapeDtypeStruct((n,), jnp.float32),
            jax.ShapeDtypeStruct((n,), jnp.int32),
        ),
        mesh=mesh,
        compiler_params=pltpu.CompilerParams(needs_layout_passes=False),
        scratch_types=[
            pltpu.VMEM((seq,), jnp.float32),
            pltpu.VMEM((seq,), jnp.int32),
            pltpu.VMEM((seq,), jnp.int32),
            pltpu.VMEM((seq,), jnp.int32),
            pltpu.VMEM((seq,), jnp.int32),
            pltpu.VMEM((_RADIX,), jnp.int32),
        ],
    )


@functools.partial(jax.jit, static_argnames=())
def kernel(x, W):
    batch, seq, d_model = x.shape
    k = max(1, int(seq * _CAPACITY))
    probs = _probs_tc(x.reshape(batch * seq, d_model), W, blk=1024)
    vals_flat, idx_flat = _make_sort_sc(batch, seq)(probs)
    vals = vals_flat.reshape(batch, seq)[:, :k]
    idx = idx_flat.reshape(batch, seq)[:, :k]
    return vals, idx, k
